# baseline (speedup 1.0000x reference)
import functools

import jax
import jax.numpy as jnp
from jax import lax
from jax.experimental import pallas as pl
from jax.experimental.pallas import tpu as pltpu
from jax.experimental.pallas import tpu_sc as plsc

_B = 8
_N = 2048
_S1 = 512
_S2 = 128
_K = 64
_LANG = 512
_R1 = 0.2
_R2 = 0.4
_NFREQ = 6


def _fps_call(n_sample, n_pts):
    rs = n_pts // 256

    def body(pos_ref, sel_ref):
        p = pos_ref[...]
        iota = (lax.broadcasted_iota(jnp.int32, (rs, 256), 0) * 256
                + lax.broadcasted_iota(jnp.int32, (rs, 256), 1))
        iota_cols = lax.broadcasted_iota(jnp.int32, (_B, n_sample), 1)

        def step(i, carry):
            last, dists, sel = carry
            oh = jnp.where(iota[None] == last[:, None, None], 1.0, 0.0)
            c = jnp.sum(p * oh[:, None], axis=(2, 3))
            diff = p - c[:, :, None, None]
            sq = diff * diff
            d = sq[:, 0] + sq[:, 1] + sq[:, 2]
            dists = jnp.minimum(dists, d)
            m = jnp.max(dists, axis=(1, 2))
            cand = jnp.where(dists == m[:, None, None], iota[None], n_pts)
            a = jnp.min(cand, axis=(1, 2))
            sel = jnp.where(iota_cols == i, a[:, None], sel)
            return a, dists, sel

        init = (jnp.zeros((_B,), jnp.int32),
                jnp.full((_B, rs, 256), jnp.inf, jnp.float32),
                jnp.zeros((_B, n_sample), jnp.int32))
        _, _, sel = lax.fori_loop(1, n_sample, step, init)
        sel_ref[...] = sel

    return pl.pallas_call(
        body,
        grid=(1,),
        in_specs=[pl.BlockSpec((_B, 3, rs, 256), lambda i: (0, 0, 0, 0))],
        out_specs=pl.BlockSpec((_B, n_sample), lambda i: (0, 0)),
        out_shape=jax.ShapeDtypeStruct((_B, n_sample), jnp.int32),
    )


def _group_call(n_sample, n_pts, radius, k):
    r2 = radius * radius

    def body(pos_ref, sel_ref, gidx_ref, valid_ref, cent_ref):
        b = pl.program_id(0)
        p = pos_ref[0]
        selc = sel_ref[0]
        iota_n = lax.broadcasted_iota(jnp.int32, (1, n_pts), 1)
        oh = jnp.where(selc == iota_n, 1.0, 0.0)
        px, py, pz = p[0:1], p[1:2], p[2:3]
        cx = jnp.sum(oh * px, axis=1, keepdims=True)
        cy = jnp.sum(oh * py, axis=1, keepdims=True)
        cz = jnp.sum(oh * pz, axis=1, keepdims=True)
        dx = cx - px
        dy = cy - py
        dz = cz - pz
        d2 = dx * dx + dy * dy + dz * dz
        neg = jnp.where(d2 <= r2, -d2, -jnp.inf)
        iota_sn = lax.broadcasted_iota(jnp.int32, (n_sample, n_pts), 1)
        iota_sk = lax.broadcasted_iota(jnp.int32, (n_sample, k), 1)

        def step(j, carry):
            neg, gidx, valid = carry
            m = jnp.max(neg, axis=1, keepdims=True)
            cand = jnp.where(neg == m, iota_sn, n_pts)
            a = jnp.min(cand, axis=1, keepdims=True)
            v = jnp.where(m > -jnp.inf, 1.0, 0.0)
            neg = jnp.where(iota_sn == a, -jnp.inf, neg)
            gidx = jnp.where(iota_sk == j, a + b * n_pts, gidx)
            valid = jnp.where(iota_sk == j, v, valid)
            return neg, gidx, valid

        init = (neg,
                jnp.zeros((n_sample, k), jnp.int32),
                jnp.zeros((n_sample, k), jnp.float32))
        _, gidx, valid = lax.fori_loop(0, k, step, init)
        gidx_ref[...] = gidx[None]
        valid_ref[...] = valid[None]
        cent_ref[...] = jnp.concatenate([cx, cy, cz], axis=1)[None]

    return pl.pallas_call(
        body,
        grid=(_B,),
        in_specs=[
            pl.BlockSpec((1, 3, n_pts), lambda b: (b, 0, 0)),
            pl.BlockSpec((1, n_sample, 1), lambda b: (b, 0, 0)),
        ],
        out_specs=[
            pl.BlockSpec((1, n_sample, k), lambda b: (b, 0, 0)),
            pl.BlockSpec((1, n_sample, k), lambda b: (b, 0, 0)),
            pl.BlockSpec((1, n_sample, 3), lambda b: (b, 0, 0)),
        ],
        out_shape=[
            jax.ShapeDtypeStruct((_B, n_sample, k), jnp.int32),
            jax.ShapeDtypeStruct((_B, n_sample, k), jnp.float32),
            jax.ShapeDtypeStruct((_B, n_sample, 3), jnp.float32),
        ],
        compiler_params=pltpu.CompilerParams(
            dimension_semantics=("parallel",)),
    )


def _sc_gather_rows(table, idx_flat, chunk):
    bn = idx_flat.shape[0]
    d = table.shape[1]
    info = plsc.get_sparse_core_info()
    nw = info.num_cores * info.num_subcores
    b_per_w = bn // nw
    n_inner = b_per_w // chunk
    mesh = plsc.VectorSubcoreMesh(core_axis_name="c", subcore_axis_name="s")

    @functools.partial(
        pl.kernel,
        mesh=mesh,
        out_type=jax.ShapeDtypeStruct((bn, d), jnp.float32),
        scratch_types=[
            pltpu.VMEM((chunk,), jnp.int32),
            pltpu.VMEM((chunk, d), jnp.float32),
            pltpu.SemaphoreType.DMA,
        ],
    )
    def k(table_hbm, idx_hbm, out_hbm, idx_v, rows_v, sem):
        wid = lax.axis_index("s") * info.num_cores + lax.axis_index("c")
        base = wid * b_per_w
        for t in range(n_inner):
            off = base + t * chunk
            pltpu.sync_copy(idx_hbm.at[pl.ds(off, chunk)], idx_v)
            pltpu.async_copy(table_hbm.at[idx_v], rows_v, sem).wait()
            pltpu.sync_copy(rows_v, out_hbm.at[pl.ds(off, chunk)])

    return k(table, idx_flat)


def _sc_gather_cols(table_flat, idx_flat, n_cols, chunk):
    bn = idx_flat.shape[0]
    per_b = bn // _B
    v_words = table_flat.shape[0]
    info = plsc.get_sparse_core_info()
    nw = info.num_cores * info.num_subcores
    b_per_w = bn // nw
    n_inner = b_per_w // chunk
    n_vec = chunk // 16
    mesh = plsc.VectorSubcoreMesh(core_axis_name="c", subcore_axis_name="s")

    @functools.partial(
        pl.kernel,
        mesh=mesh,
        out_type=jax.ShapeDtypeStruct((_B, n_cols, per_b), jnp.float32),
        scratch_types=[
            pltpu.VMEM((v_words,), jnp.float32),
            pltpu.VMEM((chunk,), jnp.int32),
            pltpu.VMEM((n_cols, chunk), jnp.float32),
        ],
        compiler_params=pltpu.CompilerParams(needs_layout_passes=False),
    )
    def k(table_hbm, idx_hbm, out_hbm, tab_v, idx_v, stage_v):
        wid = lax.axis_index("s") * info.num_cores + lax.axis_index("c")
        base = wid * b_per_w
        pltpu.sync_copy(table_hbm, tab_v)
        for t in range(n_inner):
            off = base + t * chunk
            pltpu.sync_copy(idx_hbm.at[pl.ds(off, chunk)], idx_v)

            def body(i, carry):
                iv = idx_v[pl.ds(i * 16, 16)]
                fbase = iv * n_cols
                for c in range(n_cols):
                    vals = plsc.load_gather(tab_v, [fbase + c])
                    stage_v[c, pl.ds(i * 16, 16)] = vals
                return carry

            lax.fori_loop(0, n_vec, body, 0)
            pltpu.sync_copy(
                stage_v, out_hbm.at[off // per_b, :, pl.ds(off % per_b, chunk)])

    return k(table_flat, idx_flat)


def _posenc_rows(x_t):
    cols = [x_t]
    for f in range(_NFREQ):
        s = x_t * (2.0 ** f)
        cols.append(jnp.sin(s))
        cols.append(jnp.cos(s))
    return jnp.concatenate(cols, axis=0)


def _dot_t(a, b):
    return lax.dot_general(a, b, (((0,), (0,)), ((), ())),
                           preferred_element_type=jnp.float32)


def _mlp_call(n_sample, k, c_in, cat_pos, c_out, cs):
    nchunks = n_sample // cs
    rows = cs * k

    def body(gp_ref, gf_ref, cent_ref, valid_ref, cond_ref,
             fw_ref, fb_ref, w1f_ref, w1pe_ref, w1p_ref, b1_ref,
             w2_ref, b2_ref, w3_ref, b3_ref, out_ref):
        if cat_pos:
            pos_j_t = gp_ref[0]
            feat_t = None
        else:
            g6 = gp_ref[0]
            pos_j_t = g6[0:3]
            feat_t = g6[3:3 + c_in]
        cent_t = cent_ref[0]
        rep = jnp.where(
            lax.broadcasted_iota(jnp.int32, (cs, rows), 1) // k
            == lax.broadcasted_iota(jnp.int32, (cs, rows), 0),
            1.0, 0.0)
        cent_rep = jnp.dot(cent_t, rep, preferred_element_type=jnp.float32)
        rel_t = pos_j_t - cent_rep
        pe_t = _posenc_rows(rel_t)
        h = _dot_t(pe_t, w1pe_ref[...])
        if cat_pos:
            h = h + jnp.dot(gf_ref[0], w1f_ref[...],
                            preferred_element_type=jnp.float32)
        else:
            h = h + _dot_t(feat_t, w1f_ref[...])
        h = h + _dot_t(pos_j_t, w1p_ref[...])
        h = jax.nn.relu(h + b1_ref[...])
        h = jax.nn.relu(jnp.dot(h, w2_ref[...],
                                preferred_element_type=jnp.float32) + b2_ref[...])
        h = jax.nn.relu(jnp.dot(h, w3_ref[...],
                                preferred_element_type=jnp.float32) + b3_ref[...])
        gb = jnp.dot(cond_ref[0], fw_ref[...],
                     preferred_element_type=jnp.float32) + fb_ref[...]
        gamma = gb[:, :c_out]
        beta = gb[:, c_out:]
        h = (1.0 + gamma) * h + beta
        h3 = h.reshape(cs, k, c_out)
        h3 = jnp.where(valid_ref[0][..., None] > 0.0, h3, -jnp.inf)
        hp = jnp.max(h3, axis=1)
        out_ref[...] = jnp.where(hp != -jnp.inf, hp, 0.0)[None]

    if cat_pos:
        feat_spec = pl.BlockSpec((1, rows, c_in), lambda b, c: (b, c, 0))
        d1f = (c_in, 128)
        h1 = 128
    else:
        feat_spec = pl.BlockSpec((1, 3 + c_in, rows), lambda b, c: (b, 0, c))
        d1f = (c_in, 64)
        h1 = 64
    return pl.pallas_call(
        body,
        grid=(_B, nchunks),
        in_specs=[
            feat_spec if not cat_pos
            else pl.BlockSpec((1, 3, rows), lambda b, c: (b, 0, c)),
            feat_spec,
            pl.BlockSpec((1, 3, cs), lambda b, c: (b, 0, c)),
            pl.BlockSpec((1, cs, k), lambda b, c: (b, c, 0)),
            pl.BlockSpec((1, 1, _LANG), lambda b, c: (b, 0, 0)),
            pl.BlockSpec((_LANG, 2 * c_out), lambda b, c: (0, 0)),
            pl.BlockSpec((1, 2 * c_out), lambda b, c: (0, 0)),
            pl.BlockSpec(d1f, lambda b, c: (0, 0)),
            pl.BlockSpec((39, h1), lambda b, c: (0, 0)),
            pl.BlockSpec((3, h1), lambda b, c: (0, 0)),
            pl.BlockSpec((1, h1), lambda b, c: (0, 0)),
            pl.BlockSpec((h1, h1), lambda b, c: (0, 0)),
            pl.BlockSpec((1, h1), lambda b, c: (0, 0)),
            pl.BlockSpec((h1, c_out), lambda b, c: (0, 0)),
            pl.BlockSpec((1, c_out), lambda b, c: (0, 0)),
        ],
        out_specs=pl.BlockSpec((1, cs, c_out), lambda b, c: (b, c, 0)),
        out_shape=jax.ShapeDtypeStruct((_B, n_sample, c_out), jnp.float32),
        compiler_params=pltpu.CompilerParams(
            dimension_semantics=("parallel", "parallel")),
    )


def _global_call(n_sample, c_in, c_out):
    def body(h_ref, w_ref, b_ref, out_ref):
        h = jax.nn.relu(jnp.dot(h_ref[0], w_ref[...],
                                preferred_element_type=jnp.float32) + b_ref[...])
        out_ref[...] = h[None]

    return pl.pallas_call(
        body,
        grid=(_B,),
        in_specs=[
            pl.BlockSpec((1, n_sample, c_in), lambda b: (b, 0, 0)),
            pl.BlockSpec((c_in, c_out), lambda b: (0, 0)),
            pl.BlockSpec((1, c_out), lambda b: (0, 0)),
        ],
        out_specs=pl.BlockSpec((1, n_sample, c_out), lambda b: (b, 0, 0)),
        out_shape=jax.ShapeDtypeStruct((_B, n_sample, c_out), jnp.float32),
        compiler_params=pltpu.CompilerParams(
            dimension_semantics=("parallel",)),
    )


def _posout_call(n_sample, c_out):
    def body(p_ref, w_ref, b_ref, out_ref):
        x = p_ref[0]
        cols = [x]
        for f in range(_NFREQ):
            sf = x * (2.0 ** f)
            cols.append(jnp.sin(sf))
            cols.append(jnp.cos(sf))
        pe = jnp.concatenate(cols, axis=1)
        out_ref[...] = (jnp.dot(pe, w_ref[...],
                                preferred_element_type=jnp.float32)
                        + b_ref[...])[None]

    return pl.pallas_call(
        body,
        grid=(_B,),
        in_specs=[
            pl.BlockSpec((1, n_sample, 3), lambda b: (b, 0, 0)),
            pl.BlockSpec((39, c_out), lambda b: (0, 0)),
            pl.BlockSpec((1, c_out), lambda b: (0, 0)),
        ],
        out_specs=pl.BlockSpec((1, n_sample, c_out), lambda b: (b, 0, 0)),
        out_shape=jax.ShapeDtypeStruct((_B, n_sample, c_out), jnp.float32),
        compiler_params=pltpu.CompilerParams(
            dimension_semantics=("parallel",)),
    )


def _pad_cols(x, d):
    return jnp.pad(x, ((0, 0), (0, d - x.shape[1])))


def kernel(rgb, pos, batch, cond, params):
    del batch
    pos_b = pos.reshape(_B, _N, 3)
    rgb_b = rgb.reshape(_B, _N, 3)
    pos_t = jnp.transpose(pos_b, (0, 2, 1))
    cond3 = cond.reshape(_B, 1, _LANG)

    p1 = params['sa1']
    p2 = params['sa2']

    sel1 = _fps_call(_S1, _N)(pos_t.reshape(_B, 3, _N // 256, 256))
    sel1 = sel1.reshape(_B, _S1, 1)
    gidx1, valid1, cent1 = _group_call(_S1, _N, _R1, _K)(pos_t, sel1)
    table1 = jnp.concatenate(
        [pos_b.reshape(-1, 3), rgb_b.reshape(-1, 3)], axis=1).reshape(-1)
    g1t = _sc_gather_cols(table1, gidx1.reshape(-1), 6, 2048)
    cent1_t = jnp.transpose(cent1, (0, 2, 1))
    (w1a, b1a), (w1b, b1b), (w1c, b1c) = p1['local']
    pooled1 = _mlp_call(_S1, _K, 3, False, 128, 128)(
        g1t, g1t, cent1_t, valid1, cond3,
        p1['film_W'], p1['film_b'].reshape(1, -1),
        w1a[0:3], w1a[3:42], jnp.zeros((3, 64), jnp.float32),
        b1a.reshape(1, -1), w1b, b1b.reshape(1, -1), w1c, b1c.reshape(1, -1))
    (wg1, bg1), = p1['global']
    x1 = _global_call(_S1, 128, 128)(pooled1, wg1, bg1.reshape(1, -1))

    pos1 = cent1
    pos1_t = cent1_t
    sel2 = _fps_call(_S2, _S1)(pos1_t.reshape(_B, 3, _S1 // 256, 256))
    sel2 = sel2.reshape(_B, _S2, 1)
    gidx2, valid2, cent2 = _group_call(_S2, _S1, _R2, _K)(pos1_t, sel2)
    g2pt = _sc_gather_cols(pos1.reshape(-1), gidx2.reshape(-1), 3, 2048)
    g2x = _sc_gather_rows(x1.reshape(-1, 128), gidx2.reshape(-1), 512)
    g2x = g2x.reshape(_B, _S2 * _K, 128)
    cent2_t = jnp.transpose(cent2, (0, 2, 1))
    (w2a, b2a), (w2b, b2b), (w2c, b2c) = p2['local']
    pooled2 = _mlp_call(_S2, _K, 128, True, 256, 128)(
        g2pt, g2x, cent2_t, valid2, cond3,
        p2['film_W'], p2['film_b'].reshape(1, -1),
        w2a[0:128], w2a[128:167], w2a[167:170],
        b2a.reshape(1, -1), w2b, b2b.reshape(1, -1), w2c, b2c.reshape(1, -1))
    (wg2, bg2), = p2['global']
    x2 = _global_call(_S2, 256, 256)(pooled2, wg2, bg2.reshape(1, -1))

    pos_out = _posout_call(_S2, 256)(cent2, params['pe_W'],
                                     params['pe_b'].reshape(1, -1))
    return (x2, pos_out)


# speedup vs baseline: 1.0387x; 1.0387x over previous
import functools

import jax
import jax.numpy as jnp
from jax import lax
from jax.experimental import pallas as pl
from jax.experimental.pallas import tpu as pltpu
from jax.experimental.pallas import tpu_sc as plsc

_B = 8
_N = 2048
_S1 = 512
_S2 = 128
_K = 64
_LANG = 512
_R1 = 0.2
_R2 = 0.4
_NFREQ = 6


def _fps_call(n_sample, n_pts):
    rs = n_pts // 256

    def body(pos_ref, sel_ref):
        p = pos_ref[...]
        iota = (lax.broadcasted_iota(jnp.int32, (rs, 256), 0) * 256
                + lax.broadcasted_iota(jnp.int32, (rs, 256), 1))
        iota_cols = lax.broadcasted_iota(jnp.int32, (_B, n_sample), 1)

        def step(i, carry):
            last, dists, sel = carry
            oh = jnp.where(iota[None] == last[:, None, None], 1.0, 0.0)
            c = jnp.sum(p * oh[:, None], axis=(2, 3))
            diff = p - c[:, :, None, None]
            sq = diff * diff
            d = sq[:, 0] + sq[:, 1] + sq[:, 2]
            dists = jnp.minimum(dists, d)
            m = jnp.max(dists, axis=(1, 2))
            cand = jnp.where(dists == m[:, None, None], iota[None], n_pts)
            a = jnp.min(cand, axis=(1, 2))
            sel = jnp.where(iota_cols == i, a[:, None], sel)
            return a, dists, sel

        init = (jnp.zeros((_B,), jnp.int32),
                jnp.full((_B, rs, 256), jnp.inf, jnp.float32),
                jnp.zeros((_B, n_sample), jnp.int32))
        _, _, sel = lax.fori_loop(1, n_sample, step, init)
        sel_ref[...] = sel

    return pl.pallas_call(
        body,
        grid=(1,),
        in_specs=[pl.BlockSpec((_B, 3, rs, 256), lambda i: (0, 0, 0, 0))],
        out_specs=pl.BlockSpec((_B, n_sample), lambda i: (0, 0)),
        out_shape=jax.ShapeDtypeStruct((_B, n_sample), jnp.int32),
    )


def _group_call(n_sample, n_pts, radius, k):
    r2 = radius * radius

    def body(pos_ref, sel_ref, gidx_ref, valid_ref, cent_ref):
        b = pl.program_id(0)
        p = pos_ref[0]
        selc = sel_ref[0]
        iota_n = lax.broadcasted_iota(jnp.int32, (1, n_pts), 1)
        oh = jnp.where(selc == iota_n, 1.0, 0.0)
        px, py, pz = p[0:1], p[1:2], p[2:3]
        cx = jnp.sum(oh * px, axis=1, keepdims=True)
        cy = jnp.sum(oh * py, axis=1, keepdims=True)
        cz = jnp.sum(oh * pz, axis=1, keepdims=True)
        dx = cx - px
        dy = cy - py
        dz = cz - pz
        d2 = dx * dx + dy * dy + dz * dz
        neg = jnp.where(d2 <= r2, -d2, -jnp.inf)
        iota_sn = lax.broadcasted_iota(jnp.int32, (n_sample, n_pts), 1)
        iota_sk = lax.broadcasted_iota(jnp.int32, (n_sample, k), 1)

        def step(j, carry):
            neg, gidx, valid = carry
            m = jnp.max(neg, axis=1, keepdims=True)
            cand = jnp.where(neg == m, iota_sn, n_pts)
            a = jnp.min(cand, axis=1, keepdims=True)
            v = jnp.where(m > -jnp.inf, 1.0, 0.0)
            neg = jnp.where(iota_sn == a, -jnp.inf, neg)
            gidx = jnp.where(iota_sk == j, a + b * n_pts, gidx)
            valid = jnp.where(iota_sk == j, v, valid)
            return neg, gidx, valid

        init = (neg,
                jnp.zeros((n_sample, k), jnp.int32),
                jnp.zeros((n_sample, k), jnp.float32))
        _, gidx, valid = lax.fori_loop(0, k, step, init)
        gidx_ref[...] = gidx[None]
        valid_ref[...] = valid[None]
        cent_ref[...] = jnp.concatenate([cx, cy, cz], axis=1)[None]

    return pl.pallas_call(
        body,
        grid=(_B,),
        in_specs=[
            pl.BlockSpec((1, 3, n_pts), lambda b: (b, 0, 0)),
            pl.BlockSpec((1, n_sample, 1), lambda b: (b, 0, 0)),
        ],
        out_specs=[
            pl.BlockSpec((1, n_sample, k), lambda b: (b, 0, 0)),
            pl.BlockSpec((1, n_sample, k), lambda b: (b, 0, 0)),
            pl.BlockSpec((1, n_sample, 3), lambda b: (b, 0, 0)),
        ],
        out_shape=[
            jax.ShapeDtypeStruct((_B, n_sample, k), jnp.int32),
            jax.ShapeDtypeStruct((_B, n_sample, k), jnp.float32),
            jax.ShapeDtypeStruct((_B, n_sample, 3), jnp.float32),
        ],
        compiler_params=pltpu.CompilerParams(
            dimension_semantics=("parallel",)),
    )


def _sc_gather_rows(table, idx_flat, chunk):
    bn = idx_flat.shape[0]
    d = table.shape[1]
    info = plsc.get_sparse_core_info()
    nw = info.num_cores * info.num_subcores
    b_per_w = bn // nw
    n_inner = b_per_w // chunk
    mesh = plsc.VectorSubcoreMesh(core_axis_name="c", subcore_axis_name="s")

    @functools.partial(
        pl.kernel,
        mesh=mesh,
        out_type=jax.ShapeDtypeStruct((bn, d), jnp.float32),
        scratch_types=[
            pltpu.VMEM((chunk,), jnp.int32),
            pltpu.VMEM((chunk, d), jnp.float32),
            pltpu.SemaphoreType.DMA,
        ],
    )
    def k(table_hbm, idx_hbm, out_hbm, idx_v, rows_v, sem):
        wid = lax.axis_index("s") * info.num_cores + lax.axis_index("c")
        base = wid * b_per_w
        for t in range(n_inner):
            off = base + t * chunk
            pltpu.sync_copy(idx_hbm.at[pl.ds(off, chunk)], idx_v)
            pltpu.async_copy(table_hbm.at[idx_v], rows_v, sem).wait()
            pltpu.sync_copy(rows_v, out_hbm.at[pl.ds(off, chunk)])

    return k(table, idx_flat)


def _sc_gather_cols(table_flat, idx_flat, n_cols, chunk):
    bn = idx_flat.shape[0]
    v_words = table_flat.shape[0]
    info = plsc.get_sparse_core_info()
    nw = info.num_cores * info.num_subcores
    b_per_w = bn // nw
    n_inner = b_per_w // chunk
    n_vec = chunk // 16
    mesh = plsc.VectorSubcoreMesh(core_axis_name="c", subcore_axis_name="s")

    @functools.partial(
        pl.kernel,
        mesh=mesh,
        out_type=jax.ShapeDtypeStruct((n_cols, bn), jnp.float32),
        scratch_types=[
            pltpu.VMEM((v_words,), jnp.float32),
            pltpu.VMEM((chunk,), jnp.int32),
            pltpu.VMEM((n_cols, chunk), jnp.float32),
        ],
        compiler_params=pltpu.CompilerParams(needs_layout_passes=False),
    )
    def k(table_hbm, idx_hbm, out_hbm, tab_v, idx_v, stage_v):
        wid = lax.axis_index("s") * info.num_cores + lax.axis_index("c")
        base = wid * b_per_w
        pltpu.sync_copy(table_hbm, tab_v)
        for t in range(n_inner):
            off = base + t * chunk
            pltpu.sync_copy(idx_hbm.at[pl.ds(off, chunk)], idx_v)

            def body(i, carry):
                iv = idx_v[pl.ds(i * 16, 16)]
                fbase = iv * n_cols
                for c in range(n_cols):
                    vals = plsc.load_gather(tab_v, [fbase + c])
                    stage_v[c, pl.ds(i * 16, 16)] = vals
                return carry

            lax.fori_loop(0, n_vec, body, 0)
            pltpu.sync_copy(stage_v, out_hbm.at[:, pl.ds(off, chunk)])

    return k(table_flat, idx_flat)


def _posenc_rows(x_t):
    cols = [x_t]
    for f in range(_NFREQ):
        s = x_t * (2.0 ** f)
        cols.append(jnp.sin(s))
        cols.append(jnp.cos(s))
    return jnp.concatenate(cols, axis=0)


def _dot_t(a, b):
    return lax.dot_general(a, b, (((0,), (0,)), ((), ())),
                           preferred_element_type=jnp.float32)


def _mlp_call(n_sample, k, c_in, cat_pos, c_out, cs):
    nchunks = n_sample // cs
    rows = cs * k

    def body(gp_ref, gf_ref, cent_ref, valid_ref, cond_ref,
             fw_ref, fb_ref, w1f_ref, w1pe_ref, w1p_ref, b1_ref,
             w2_ref, b2_ref, w3_ref, b3_ref, out_ref):
        pos_j_t = gp_ref[0]
        cent_t = cent_ref[0]
        rep = jnp.where(
            lax.broadcasted_iota(jnp.int32, (cs, rows), 1) // k
            == lax.broadcasted_iota(jnp.int32, (cs, rows), 0),
            1.0, 0.0)
        cent_rep = jnp.dot(cent_t, rep, preferred_element_type=jnp.float32)
        rel_t = pos_j_t - cent_rep
        pe_t = _posenc_rows(rel_t)
        h = _dot_t(pe_t, w1pe_ref[...])
        if cat_pos:
            h = h + jnp.dot(gf_ref[0], w1f_ref[...],
                            preferred_element_type=jnp.float32)
        else:
            h = h + _dot_t(gf_ref[0], w1f_ref[...])
        h = h + _dot_t(pos_j_t, w1p_ref[...])
        h = jax.nn.relu(h + b1_ref[...])
        h = jax.nn.relu(jnp.dot(h, w2_ref[...],
                                preferred_element_type=jnp.float32) + b2_ref[...])
        h = jax.nn.relu(jnp.dot(h, w3_ref[...],
                                preferred_element_type=jnp.float32) + b3_ref[...])
        gb = jnp.dot(cond_ref[0], fw_ref[...],
                     preferred_element_type=jnp.float32) + fb_ref[...]
        gamma = gb[:, :c_out]
        beta = gb[:, c_out:]
        h = (1.0 + gamma) * h + beta
        h3 = h.reshape(cs, k, c_out)
        h3 = jnp.where(valid_ref[0][..., None] > 0.0, h3, -jnp.inf)
        hp = jnp.max(h3, axis=1)
        out_ref[...] = jnp.where(hp != -jnp.inf, hp, 0.0)[None]

    if cat_pos:
        feat_spec = pl.BlockSpec((1, rows, c_in), lambda b, c: (b, c, 0))
        d1f = (c_in, 128)
        h1 = 128
    else:
        feat_spec = pl.BlockSpec((1, c_in, rows), lambda b, c: (b, 0, c))
        d1f = (c_in, 64)
        h1 = 64
    return pl.pallas_call(
        body,
        grid=(_B, nchunks),
        in_specs=[
            pl.BlockSpec((1, 3, rows), lambda b, c: (b, 0, c)),
            feat_spec,
            pl.BlockSpec((1, 3, cs), lambda b, c: (b, 0, c)),
            pl.BlockSpec((1, cs, k), lambda b, c: (b, c, 0)),
            pl.BlockSpec((1, 1, _LANG), lambda b, c: (b, 0, 0)),
            pl.BlockSpec((_LANG, 2 * c_out), lambda b, c: (0, 0)),
            pl.BlockSpec((1, 2 * c_out), lambda b, c: (0, 0)),
            pl.BlockSpec(d1f, lambda b, c: (0, 0)),
            pl.BlockSpec((39, h1), lambda b, c: (0, 0)),
            pl.BlockSpec((3, h1), lambda b, c: (0, 0)),
            pl.BlockSpec((1, h1), lambda b, c: (0, 0)),
            pl.BlockSpec((h1, h1), lambda b, c: (0, 0)),
            pl.BlockSpec((1, h1), lambda b, c: (0, 0)),
            pl.BlockSpec((h1, c_out), lambda b, c: (0, 0)),
            pl.BlockSpec((1, c_out), lambda b, c: (0, 0)),
        ],
        out_specs=pl.BlockSpec((1, cs, c_out), lambda b, c: (b, c, 0)),
        out_shape=jax.ShapeDtypeStruct((_B, n_sample, c_out), jnp.float32),
        compiler_params=pltpu.CompilerParams(
            dimension_semantics=("parallel", "parallel")),
    )


def _global_call(n_sample, c_in, c_out):
    def body(h_ref, w_ref, b_ref, out_ref):
        h = jax.nn.relu(jnp.dot(h_ref[0], w_ref[...],
                                preferred_element_type=jnp.float32) + b_ref[...])
        out_ref[...] = h[None]

    return pl.pallas_call(
        body,
        grid=(_B,),
        in_specs=[
            pl.BlockSpec((1, n_sample, c_in), lambda b: (b, 0, 0)),
            pl.BlockSpec((c_in, c_out), lambda b: (0, 0)),
            pl.BlockSpec((1, c_out), lambda b: (0, 0)),
        ],
        out_specs=pl.BlockSpec((1, n_sample, c_out), lambda b: (b, 0, 0)),
        out_shape=jax.ShapeDtypeStruct((_B, n_sample, c_out), jnp.float32),
        compiler_params=pltpu.CompilerParams(
            dimension_semantics=("parallel",)),
    )


def _posout_call(n_sample, c_out):
    def body(p_ref, w_ref, b_ref, out_ref):
        x = p_ref[0]
        cols = [x]
        for f in range(_NFREQ):
            sf = x * (2.0 ** f)
            cols.append(jnp.sin(sf))
            cols.append(jnp.cos(sf))
        pe = jnp.concatenate(cols, axis=1)
        out_ref[...] = (jnp.dot(pe, w_ref[...],
                                preferred_element_type=jnp.float32)
                        + b_ref[...])[None]

    return pl.pallas_call(
        body,
        grid=(_B,),
        in_specs=[
            pl.BlockSpec((1, n_sample, 3), lambda b: (b, 0, 0)),
            pl.BlockSpec((39, c_out), lambda b: (0, 0)),
            pl.BlockSpec((1, c_out), lambda b: (0, 0)),
        ],
        out_specs=pl.BlockSpec((1, n_sample, c_out), lambda b: (b, 0, 0)),
        out_shape=jax.ShapeDtypeStruct((_B, n_sample, c_out), jnp.float32),
        compiler_params=pltpu.CompilerParams(
            dimension_semantics=("parallel",)),
    )


def _pad_cols(x, d):
    return jnp.pad(x, ((0, 0), (0, d - x.shape[1])))


def kernel(rgb, pos, batch, cond, params):
    del batch
    pos_b = pos.reshape(_B, _N, 3)
    rgb_b = rgb.reshape(_B, _N, 3)
    pos_t = jnp.transpose(pos_b, (0, 2, 1))
    cond3 = cond.reshape(_B, 1, _LANG)

    p1 = params['sa1']
    p2 = params['sa2']

    sel1 = _fps_call(_S1, _N)(pos_t.reshape(_B, 3, _N // 256, 256))
    sel1 = sel1.reshape(_B, _S1, 1)
    gidx1, valid1, cent1 = _group_call(_S1, _N, _R1, _K)(pos_t, sel1)
    table1 = jnp.concatenate(
        [pos_b.reshape(-1, 3), rgb_b.reshape(-1, 3)], axis=1).reshape(-1)
    g1t = _sc_gather_cols(table1, gidx1.reshape(-1), 6, 2048)
    g1t = jnp.transpose(g1t.reshape(6, _B, _S1 * _K), (1, 0, 2))
    cent1_t = jnp.transpose(cent1, (0, 2, 1))
    (w1a, b1a), (w1b, b1b), (w1c, b1c) = p1['local']
    pooled1 = _mlp_call(_S1, _K, 3, False, 128, 128)(
        g1t[:, 0:3], g1t[:, 3:6], cent1_t, valid1, cond3,
        p1['film_W'], p1['film_b'].reshape(1, -1),
        w1a[0:3], w1a[3:42], jnp.zeros((3, 64), jnp.float32),
        b1a.reshape(1, -1), w1b, b1b.reshape(1, -1), w1c, b1c.reshape(1, -1))
    (wg1, bg1), = p1['global']
    x1 = _global_call(_S1, 128, 128)(pooled1, wg1, bg1.reshape(1, -1))

    pos1 = cent1
    pos1_t = cent1_t
    sel2 = _fps_call(_S2, _S1)(pos1_t.reshape(_B, 3, _S1 // 256, 256))
    sel2 = sel2.reshape(_B, _S2, 1)
    gidx2, valid2, cent2 = _group_call(_S2, _S1, _R2, _K)(pos1_t, sel2)
    g2pt = _sc_gather_cols(pos1.reshape(-1), gidx2.reshape(-1), 3, 2048)
    g2pt = jnp.transpose(g2pt.reshape(3, _B, _S2 * _K), (1, 0, 2))
    g2x = _sc_gather_rows(x1.reshape(-1, 128), gidx2.reshape(-1), 512)
    g2x = g2x.reshape(_B, _S2 * _K, 128)
    cent2_t = jnp.transpose(cent2, (0, 2, 1))
    (w2a, b2a), (w2b, b2b), (w2c, b2c) = p2['local']
    pooled2 = _mlp_call(_S2, _K, 128, True, 256, 128)(
        g2pt, g2x, cent2_t, valid2, cond3,
        p2['film_W'], p2['film_b'].reshape(1, -1),
        w2a[0:128], w2a[128:167], w2a[167:170],
        b2a.reshape(1, -1), w2b, b2b.reshape(1, -1), w2c, b2c.reshape(1, -1))
    (wg2, bg2), = p2['global']
    x2 = _global_call(_S2, 256, 256)(pooled2, wg2, bg2.reshape(1, -1))

    pos_out = _posout_call(_S2, 256)(cent2, params['pe_W'],
                                     params['pe_b'].reshape(1, -1))
    return (x2, pos_out)
